# baseline (speedup 1.0000x reference)
import functools

import jax
import jax.numpy as jnp
from jax.experimental import pallas as pl

N = 196
K = 49
B = 64


def _body(x_ref, nrow_ref, ncol_ref, out_ref, shuf_ref, rank_ref):
    nrow = nrow_ref[0]
    ncol = ncol_ref[0]
    ii = jax.lax.broadcasted_iota(jnp.int32, (N, N), 0)
    jj = jax.lax.broadcasted_iota(jnp.int32, (N, N), 1)
    lt = nrow < ncol
    eq = nrow == ncol
    hit = (lt | (eq & (jj < ii))).astype(jnp.int32)
    rank_col = jnp.sum(hit, axis=1, keepdims=True)
    rank_row = 195 - jnp.sum(hit, axis=0, keepdims=True)
    rank_ref[0] = rank_col

    rr = jax.lax.broadcasted_iota(jnp.int32, (N, 1), 0)
    eqm = (rank_row == rr)
    irow = jax.lax.broadcasted_iota(jnp.int32, (1, N), 1)
    shuf_col = jnp.sum(eqm.astype(jnp.int32) * irow, axis=1, keepdims=True)
    shuf_ref[0] = shuf_col

    onehot = (shuf_col[:K] == irow).astype(jnp.float32)
    val = x_ref[0, 1:, :]
    out_ref[0, 0, :] = x_ref[0, 0, :]
    out_ref[0, pl.ds(1, K), :] = jnp.dot(
        onehot, val, preferred_element_type=jnp.float32)


@jax.jit
def kernel(x, noise):
    noise_row = noise.reshape(B, 1, N)
    noise_col = noise.reshape(B, N, 1)
    out, shuf, rank = pl.pallas_call(
        _body,
        grid=(B,),
        in_specs=[
            pl.BlockSpec((1, 197, 768), lambda b: (b, 0, 0)),
            pl.BlockSpec((1, 1, N), lambda b: (b, 0, 0)),
            pl.BlockSpec((1, N, 1), lambda b: (b, 0, 0)),
        ],
        out_specs=[
            pl.BlockSpec((1, K + 1, 768), lambda b: (b, 0, 0)),
            pl.BlockSpec((1, N, 1), lambda b: (b, 0, 0)),
            pl.BlockSpec((1, N, 1), lambda b: (b, 0, 0)),
        ],
        out_shape=[
            jax.ShapeDtypeStruct((B, K + 1, 768), jnp.float32),
            jax.ShapeDtypeStruct((B, N, 1), jnp.int32),
            jax.ShapeDtypeStruct((B, N, 1), jnp.int32),
        ],
    )(x, noise_row, noise_col)
    shuf = shuf.reshape(B, N)
    rank = rank.reshape(B, N)
    remain_idx = shuf[:, :K]
    masked_idx = shuf[:, K:]
    remain_padding_mask = jnp.ones((B, K + 1), dtype=jnp.float32)
    revert_padding_mask = jnp.ones((B, N + 1), dtype=jnp.float32)
    return (out, remain_idx, masked_idx, rank,
            remain_padding_mask, revert_padding_mask)
